# baseline (speedup 1.0000x reference)
import functools

import jax
import jax.numpy as jnp
from jax.experimental import pallas as pl

_K = 16


def _body(xr_ref, xa_ref, xat_ref, wl_ref, bl_ref, wc_ref, bc_ref, o_ref):
    xr = xr_ref[0]
    xa = xa_ref[0]
    xat = xat_ref[0]

    xx = jnp.sum(xa * xa, axis=1)
    scores = 2.0 * jax.lax.dot_general(
        xr, xat, (((1,), (0,)), ((), ())),
        preferred_element_type=jnp.float32) - xx[None, :]

    iota = jax.lax.broadcasted_iota(jnp.int32, scores.shape, 1)
    big = jnp.int32(2**30)
    neg = jnp.float32(-1e30)
    r0 = pl.program_id(1) * xr.shape[0]
    rowi = jax.lax.broadcasted_iota(jnp.int32, scores.shape, 0) + r0
    scores = jnp.where(iota == rowi, neg, scores)
    h = xr
    for _ in range(_K - 1):
        sel = jnp.argmax(scores, axis=1).astype(jnp.int32)[:, None]
        onehot = (iota == sel)
        g = jax.lax.dot_general(
            onehot.astype(jnp.float32), xa, (((1,), (0,)), ((), ())),
            preferred_element_type=jnp.float32)
        h = jnp.maximum(h, g)
        scores = jnp.where(onehot, neg, scores)

    w = jax.lax.dot_general(wl_ref[...], wc_ref[...],
                            (((1,), (0,)), ((), ())),
                            preferred_element_type=jnp.float32)
    bias = jax.lax.dot_general(bl_ref[...], wc_ref[...],
                               (((1,), (0,)), ((), ())),
                               preferred_element_type=jnp.float32) + bc_ref[...]
    out = jax.lax.dot_general(h, w, (((1,), (0,)), ((), ())),
                              preferred_element_type=jnp.float32) + bias
    o_ref[0] = jnp.maximum(out, 0.0)


@functools.partial(jax.jit, static_argnames=())
def kernel(x, W_lin, b_lin, W_conv, b_conv):
    B, N, C = x.shape
    R = 512
    out_f = W_conv.shape[1]
    xt = jnp.swapaxes(x, 1, 2)
    bl = b_lin.reshape(1, -1)
    bc = b_conv.reshape(1, -1)
    grid = (B, N // R)
    return pl.pallas_call(
        _body,
        grid=grid,
        in_specs=[
            pl.BlockSpec((1, R, C), lambda b, i: (b, i, 0)),
            pl.BlockSpec((1, N, C), lambda b, i: (b, 0, 0)),
            pl.BlockSpec((1, C, N), lambda b, i: (b, 0, 0)),
            pl.BlockSpec((C, W_lin.shape[1]), lambda b, i: (0, 0)),
            pl.BlockSpec((1, b_lin.shape[0]), lambda b, i: (0, 0)),
            pl.BlockSpec((W_conv.shape[0], out_f), lambda b, i: (0, 0)),
            pl.BlockSpec((1, out_f), lambda b, i: (0, 0)),
        ],
        out_specs=pl.BlockSpec((1, R, out_f), lambda b, i: (b, i, 0)),
        out_shape=jax.ShapeDtypeStruct((B, N, out_f), jnp.float32),
    )(x, x, xt, W_lin, bl, W_conv, bc)


# speedup vs baseline: 1.0631x; 1.0631x over previous
import functools

import jax
import jax.numpy as jnp
from jax.experimental import pallas as pl

_K = 16


def _body(xr_ref, xa_ref, xat_ref, wl_ref, bl_ref, wc_ref, bc_ref, o_ref):
    xr = xr_ref[0]
    xa = xa_ref[0]
    xat = xat_ref[0]

    xx = jnp.sum(xa * xa, axis=1)
    scores = 2.0 * jax.lax.dot_general(
        xr, xat, (((1,), (0,)), ((), ())),
        preferred_element_type=jnp.float32) - xx[None, :]

    iota = jax.lax.broadcasted_iota(jnp.int32, scores.shape, 1)
    big = jnp.int32(2**30)
    neg = jnp.float32(-1e30)
    r0 = pl.program_id(1) * xr.shape[0]
    rowi = jax.lax.broadcasted_iota(jnp.int32, scores.shape, 0) + r0
    scores = jnp.where(iota == rowi, neg, scores)
    h = xr
    for _ in range(_K - 1):
        sel = jnp.argmax(scores, axis=1).astype(jnp.int32)[:, None]
        onehot = (iota == sel)
        g = jax.lax.dot_general(
            onehot.astype(jnp.float32), xa, (((1,), (0,)), ((), ())),
            preferred_element_type=jnp.float32)
        h = jnp.maximum(h, g)
        scores = jnp.where(onehot, neg, scores)

    w = jax.lax.dot_general(wl_ref[...], wc_ref[...],
                            (((1,), (0,)), ((), ())),
                            preferred_element_type=jnp.float32)
    bias = jax.lax.dot_general(bl_ref[...], wc_ref[...],
                               (((1,), (0,)), ((), ())),
                               preferred_element_type=jnp.float32) + bc_ref[...]
    out = jax.lax.dot_general(h, w, (((1,), (0,)), ((), ())),
                              preferred_element_type=jnp.float32) + bias
    o_ref[0] = jnp.maximum(out, 0.0)


@functools.partial(jax.jit, static_argnames=())
def kernel(x, W_lin, b_lin, W_conv, b_conv):
    B, N, C = x.shape
    R = 128
    out_f = W_conv.shape[1]
    xt = jnp.swapaxes(x, 1, 2)
    bl = b_lin.reshape(1, -1)
    bc = b_conv.reshape(1, -1)
    grid = (B, N // R)
    return pl.pallas_call(
        _body,
        grid=grid,
        in_specs=[
            pl.BlockSpec((1, R, C), lambda b, i: (b, i, 0)),
            pl.BlockSpec((1, N, C), lambda b, i: (b, 0, 0)),
            pl.BlockSpec((1, C, N), lambda b, i: (b, 0, 0)),
            pl.BlockSpec((C, W_lin.shape[1]), lambda b, i: (0, 0)),
            pl.BlockSpec((1, b_lin.shape[0]), lambda b, i: (0, 0)),
            pl.BlockSpec((W_conv.shape[0], out_f), lambda b, i: (0, 0)),
            pl.BlockSpec((1, out_f), lambda b, i: (0, 0)),
        ],
        out_specs=pl.BlockSpec((1, R, out_f), lambda b, i: (b, i, 0)),
        out_shape=jax.ShapeDtypeStruct((B, N, out_f), jnp.float32),
    )(x, x, xt, W_lin, bl, W_conv, bc)


# speedup vs baseline: 1.0823x; 1.0180x over previous
import functools

import jax
import jax.numpy as jnp
from jax.experimental import pallas as pl

_K = 16


def _body(xr_ref, xa_ref, xat_ref, wl_ref, bl_ref, wc_ref, bc_ref, o_ref):
    xr = xr_ref[0]
    xa = xa_ref[0]
    xat = xat_ref[0]

    xx = jnp.sum(xa * xa, axis=1)
    scores = 2.0 * jax.lax.dot_general(
        xr, xat, (((1,), (0,)), ((), ())),
        preferred_element_type=jnp.float32) - xx[None, :]

    iota = jax.lax.broadcasted_iota(jnp.int32, scores.shape, 1)
    big = jnp.int32(2**30)
    neg = jnp.float32(-1e30)
    r0 = pl.program_id(1) * xr.shape[0]
    rowi = jax.lax.broadcasted_iota(jnp.int32, scores.shape, 0) + r0
    scores = jnp.where(iota == rowi, neg, scores)
    h = xr
    xab = xa.astype(jnp.bfloat16)
    one_b = jnp.bfloat16(1)
    zero_b = jnp.bfloat16(0)
    for _ in range(_K - 1):
        sel = jnp.argmax(scores, axis=1).astype(jnp.int32)[:, None]
        onehot = (iota == sel)
        g = jax.lax.dot_general(
            onehot.astype(jnp.float32).astype(jnp.bfloat16), xab,
            (((1,), (0,)), ((), ())),
            preferred_element_type=jnp.float32)
        h = jnp.maximum(h, g)
        scores = jnp.where(onehot, neg, scores)

    w = jax.lax.dot_general(wl_ref[...], wc_ref[...],
                            (((1,), (0,)), ((), ())),
                            preferred_element_type=jnp.float32)
    bias = jax.lax.dot_general(bl_ref[...], wc_ref[...],
                               (((1,), (0,)), ((), ())),
                               preferred_element_type=jnp.float32) + bc_ref[...]
    out = jax.lax.dot_general(h, w, (((1,), (0,)), ((), ())),
                              preferred_element_type=jnp.float32) + bias
    o_ref[0] = jnp.maximum(out, 0.0)


@functools.partial(jax.jit, static_argnames=())
def kernel(x, W_lin, b_lin, W_conv, b_conv):
    B, N, C = x.shape
    R = 256
    out_f = W_conv.shape[1]
    xt = jnp.swapaxes(x, 1, 2)
    bl = b_lin.reshape(1, -1)
    bc = b_conv.reshape(1, -1)
    grid = (B, N // R)
    return pl.pallas_call(
        _body,
        grid=grid,
        in_specs=[
            pl.BlockSpec((1, R, C), lambda b, i: (b, i, 0)),
            pl.BlockSpec((1, N, C), lambda b, i: (b, 0, 0)),
            pl.BlockSpec((1, C, N), lambda b, i: (b, 0, 0)),
            pl.BlockSpec((C, W_lin.shape[1]), lambda b, i: (0, 0)),
            pl.BlockSpec((1, b_lin.shape[0]), lambda b, i: (0, 0)),
            pl.BlockSpec((W_conv.shape[0], out_f), lambda b, i: (0, 0)),
            pl.BlockSpec((1, out_f), lambda b, i: (0, 0)),
        ],
        out_specs=pl.BlockSpec((1, R, out_f), lambda b, i: (b, i, 0)),
        out_shape=jax.ShapeDtypeStruct((B, N, out_f), jnp.float32),
    )(x, x, xt, W_lin, bl, W_conv, bc)


# speedup vs baseline: 1.3400x; 1.2381x over previous
import functools

import jax
import jax.numpy as jnp
from jax.experimental import pallas as pl

_K = 16


def _body(xr_ref, xa_ref, wl_ref, bl_ref, wc_ref, bc_ref, o_ref):
    xr = xr_ref[0]
    xa = xa_ref[0]

    xx = jnp.sum(xa * xa, axis=1)
    st = 2.0 * jax.lax.dot_general(
        xa, xr, (((1,), (1,)), ((), ())),
        preferred_element_type=jnp.float32) - xx[:, None]

    iota = jax.lax.broadcasted_iota(jnp.int32, st.shape, 0)
    neg = jnp.float32(-1e30)
    r0 = pl.program_id(1) * xr.shape[0]
    rowi = jax.lax.broadcasted_iota(jnp.int32, st.shape, 1) + r0
    st = jnp.where(iota == rowi, neg, st)
    h = xr
    for _ in range(_K - 1):
        sel = jnp.argmax(st, axis=0).astype(jnp.int32)[None, :]
        onehot = (iota == sel)
        g = jax.lax.dot_general(
            onehot.astype(jnp.float32), xa, (((0,), (0,)), ((), ())),
            preferred_element_type=jnp.float32)
        h = jnp.maximum(h, g)
        st = jnp.where(onehot, neg, st)

    w = jax.lax.dot_general(wl_ref[...], wc_ref[...],
                            (((1,), (0,)), ((), ())),
                            preferred_element_type=jnp.float32)
    bias = jax.lax.dot_general(bl_ref[...], wc_ref[...],
                               (((1,), (0,)), ((), ())),
                               preferred_element_type=jnp.float32) + bc_ref[...]
    out = jax.lax.dot_general(h, w, (((1,), (0,)), ((), ())),
                              preferred_element_type=jnp.float32) + bias
    o_ref[0] = jnp.maximum(out, 0.0)


@functools.partial(jax.jit, static_argnames=())
def kernel(x, W_lin, b_lin, W_conv, b_conv):
    B, N, C = x.shape
    R = 256
    out_f = W_conv.shape[1]
    bl = b_lin.reshape(1, -1)
    bc = b_conv.reshape(1, -1)
    grid = (B, N // R)
    return pl.pallas_call(
        _body,
        grid=grid,
        in_specs=[
            pl.BlockSpec((1, R, C), lambda b, i: (b, i, 0)),
            pl.BlockSpec((1, N, C), lambda b, i: (b, 0, 0)),
            pl.BlockSpec((C, W_lin.shape[1]), lambda b, i: (0, 0)),
            pl.BlockSpec((1, b_lin.shape[0]), lambda b, i: (0, 0)),
            pl.BlockSpec((W_conv.shape[0], out_f), lambda b, i: (0, 0)),
            pl.BlockSpec((1, out_f), lambda b, i: (0, 0)),
        ],
        out_specs=pl.BlockSpec((1, R, out_f), lambda b, i: (b, i, 0)),
        out_shape=jax.ShapeDtypeStruct((B, N, out_f), jnp.float32),
    )(x, x, W_lin, bl, W_conv, bc)
